# baseline (speedup 1.0000x reference)
import jax
import jax.numpy as jnp
from jax import lax
from jax.experimental import pallas as pl
from jax.experimental.pallas import tpu as pltpu
from jax.experimental.pallas import tpu_sc as plsc

_VOCAB = 1000000
_H = 64
_CTX = 200
_NCORES = 2
_NSUB = 16
_IDS_PER_W = 8
_ACTIVE_W = 25
_MV_BLOCK = 16384


def _sc_gather_body(ids_hbm, table_hbm, out_hbm, idx_v, rows_v, acc_v, sem):
    wid = lax.axis_index("s") * _NCORES + lax.axis_index("c")

    @pl.when(wid < _ACTIVE_W)
    def _():
        base = wid * _IDS_PER_W
        pltpu.sync_copy(ids_hbm.at[pl.ds(base, _IDS_PER_W)], idx_v)
        pltpu.async_copy(table_hbm.at[idx_v], rows_v, sem).wait()
        for c in range(_H // 16):
            acc = rows_v[0, pl.ds(c * 16, 16)]
            for j in range(1, _IDS_PER_W):
                acc = acc + rows_v[j, pl.ds(c * 16, 16)]
            acc_v[pl.ds(c * 16, 16)] = acc
        pltpu.sync_copy(acc_v, out_hbm.at[wid])


def _sc_gather(ids, token_table):
    return pl.kernel(
        _sc_gather_body,
        out_type=jax.ShapeDtypeStruct((_ACTIVE_W, _H), jnp.float32),
        mesh=plsc.VectorSubcoreMesh(
            core_axis_name="c", subcore_axis_name="s",
            num_cores=_NCORES, num_subcores=_NSUB),
        scratch_types=[
            pltpu.VMEM((_IDS_PER_W,), jnp.int32),
            pltpu.VMEM((_IDS_PER_W, _H), jnp.float32),
            pltpu.VMEM((_H,), jnp.float32),
            pltpu.SemaphoreType.DMA,
        ],
        compiler_params=pltpu.CompilerParams(use_tc_tiling_on_sc=False),
    )(ids, token_table)


def _head_body(nidx_ref, pidx_ref, lidx_ref, part_ref, node_ref, par_ref,
               lang_ref, nf_ref, numw_ref, numb_ref, hidw_ref, hidb_ref,
               sww_ref, swb_ref, hid_out, sw_out):
    tok = jnp.sum(part_ref[...], axis=0, keepdims=True) * (1.0 / _CTX)
    ni = nidx_ref[0]
    pi = pidx_ref[0]
    li = lidx_ref[0]
    feat = (node_ref[pl.ds(ni, 1), :]
            + par_ref[pl.ds(pi, 1), :]
            + lang_ref[pl.ds(li, 1), :])
    nproj = lax.dot_general(nf_ref[...], numw_ref[...], (((1,), (1,)), ((), ())),
                            preferred_element_type=jnp.float32)
    feat = feat + nproj + numb_ref[...]
    cat = jnp.concatenate([tok, feat], axis=1)
    hid = jnp.tanh(
        lax.dot_general(cat, hidw_ref[...], (((1,), (1,)), ((), ())),
                        preferred_element_type=jnp.float32)
        + hidb_ref[...])
    sw = jnp.sum(hid * sww_ref[...], axis=1, keepdims=True) + swb_ref[0]
    hid_out[...] = hid
    sw_out[...] = sw


def _head(nidx, pidx, lidx, part, node_table, parent_table, lang_table,
          nf, num_w, num_b, hid_w, hid_b, sw_w, sw_b):
    smem = pl.BlockSpec(memory_space=pltpu.SMEM)
    vmem = pl.BlockSpec(memory_space=pltpu.VMEM)
    return pl.pallas_call(
        _head_body,
        in_specs=[smem, smem, smem] + [vmem] * 10 + [smem],
        out_shape=(jax.ShapeDtypeStruct((1, _H), jnp.float32),
                   jax.ShapeDtypeStruct((1, 1), jnp.float32)),
    )(nidx, pidx, lidx, part, node_table, parent_table, lang_table,
      nf, num_w, num_b, hid_w, hid_b, sw_w, sw_b)


def _mv_body(h_ref, w_ref, b_ref, o_ref):
    o_ref[...] = (lax.dot_general(h_ref[...], w_ref[...],
                                  (((1,), (1,)), ((), ())),
                                  preferred_element_type=jnp.float32)
                  + b_ref[...])


def _matvec(hidden, pref_w, pref_b):
    grid = pl.cdiv(_VOCAB, _MV_BLOCK)
    out = pl.pallas_call(
        _mv_body,
        grid=(grid,),
        in_specs=[
            pl.BlockSpec((1, _H), lambda i: (0, 0)),
            pl.BlockSpec((_MV_BLOCK, _H), lambda i: (i, 0)),
            pl.BlockSpec((1, _MV_BLOCK), lambda i: (0, i)),
        ],
        out_specs=pl.BlockSpec((1, _MV_BLOCK), lambda i: (0, i)),
        out_shape=jax.ShapeDtypeStruct((1, _VOCAB), jnp.float32),
    )(hidden, pref_w, pref_b.reshape(1, _VOCAB))
    return out.reshape(_VOCAB)


def kernel(prefix_ids, node_idx, parent_idx, lang_idx, numeric_features,
           token_table, node_table, parent_table, lang_table,
           num_W, num_b, hid_W, hid_b, sw_W, sw_b, pref_W, pref_b):
    ids = prefix_ids[-_CTX:].astype(jnp.int32)
    part = _sc_gather(ids, token_table)
    nidx = jnp.asarray(node_idx, jnp.int32).reshape(1)
    pidx = jnp.asarray(parent_idx, jnp.int32).reshape(1)
    lidx = jnp.asarray(lang_idx, jnp.int32).reshape(1)
    hidden, sw = _head(
        nidx, pidx, lidx, part, node_table, parent_table, lang_table,
        numeric_features.reshape(1, 3), num_W, num_b.reshape(1, _H),
        hid_W, hid_b.reshape(1, _H), sw_W, sw_b.reshape(1))
    logits = _matvec(hidden, pref_W, pref_b)
    return sw[0, 0], logits
